# baseline (speedup 1.0000x reference)
import jax
import jax.numpy as jnp
from jax import lax
from jax.experimental import pallas as pl
from jax.experimental.pallas import tpu as pltpu
from jax.experimental.pallas import tpu_sc as plsc

BATCH = 4
SEQ = 2048
EMBED = 1024
N = BATCH * SEQ

NUM_CORES = 2
NUM_SUBCORES = 16
NW = NUM_CORES * NUM_SUBCORES
POS_PER_W = SEQ // NW
ROWS_PER_W = POS_PER_W * BATCH
CHUNK = 16
CHUNKS_PER_B = POS_PER_W // CHUNK
NCHUNK = CHUNKS_PER_B * BATCH
LANES = 16
VECS_PER_ROW = EMBED // LANES


NBUF = 3


def _sc_body(x_hbm, tok_hbm, pos_hbm, out_hbm,
             idx_v, pos_v, tok0, tok1, tok2, g0, g1, g2, o0, o1, o2, psem):
    wid = lax.axis_index("s") * NUM_CORES + lax.axis_index("c")
    pos0 = wid * POS_PER_W

    for b in range(BATCH):
        pltpu.sync_copy(x_hbm.at[pl.ds(b * SEQ + pos0, POS_PER_W)],
                        idx_v.at[pl.ds(b * POS_PER_W, POS_PER_W)])

    pos_cp = pltpu.async_copy(pos_hbm.at[pl.ds(pos0, POS_PER_W)], pos_v, psem)

    bufs = (tok0, tok1, tok2)
    gsems = (g0, g1, g2)
    osems = (o0, o1, o2)

    def idx_slice(c):
        return idx_v.at[pl.ds(c * CHUNK, CHUNK)]

    def out_slice(c):
        b, sub = c // CHUNKS_PER_B, c % CHUNKS_PER_B
        return out_hbm.at[pl.ds(b * SEQ + pos0 + sub * CHUNK, CHUNK)]

    gathers = [None] * NCHUNK
    scatters = [None] * NCHUNK
    for c in range(NBUF - 1):
        gathers[c] = pltpu.async_copy(
            tok_hbm.at[idx_slice(c)], bufs[c % NBUF], gsems[c % NBUF])
    pos_cp.wait()

    for c in range(NCHUNK):
        k = c % NBUF
        buf = bufs[k]
        gathers[c].wait()
        nxt = c + NBUF - 1
        if nxt < NCHUNK:
            if nxt >= NBUF:
                scatters[nxt - NBUF].wait()
            gathers[nxt] = pltpu.async_copy(
                tok_hbm.at[idx_slice(nxt)], bufs[nxt % NBUF], gsems[nxt % NBUF])

        prow = (c % CHUNKS_PER_B) * CHUNK

        def add_row(r, carry):
            for j in range(VECS_PER_ROW):
                sl = pl.ds(j * LANES, LANES)
                plsc.addupdate(buf.at[r, sl], pos_v[prow + r, sl])
            return carry

        lax.fori_loop(0, CHUNK, add_row, 0)

        scatters[c] = pltpu.async_copy(buf, out_slice(c), osems[k])

    for c in range(NCHUNK - min(NBUF, NCHUNK), NCHUNK):
        scatters[c].wait()


@jax.jit
def kernel(x, token_table, pos_table):
    xf = x.reshape(-1).astype(jnp.int32)
    mesh = plsc.VectorSubcoreMesh(
        core_axis_name="c", subcore_axis_name="s",
        num_cores=NUM_CORES, num_subcores=NUM_SUBCORES,
    )
    out_flat = pl.kernel(
        _sc_body,
        out_type=jax.ShapeDtypeStruct((N, EMBED), jnp.float32),
        mesh=mesh,
        scratch_types=[
            pltpu.VMEM((ROWS_PER_W,), jnp.int32),
            pltpu.VMEM((POS_PER_W, EMBED), jnp.float32),
            pltpu.VMEM((CHUNK, EMBED), jnp.float32),
            pltpu.VMEM((CHUNK, EMBED), jnp.float32),
            pltpu.VMEM((CHUNK, EMBED), jnp.float32),
            pltpu.SemaphoreType.DMA,
            pltpu.SemaphoreType.DMA,
            pltpu.SemaphoreType.DMA,
            pltpu.SemaphoreType.DMA,
            pltpu.SemaphoreType.DMA,
            pltpu.SemaphoreType.DMA,
            pltpu.SemaphoreType.DMA,
        ],
    )(xf, token_table, pos_table)
    return out_flat.reshape(BATCH, SEQ, EMBED)


# speedup vs baseline: 1.0029x; 1.0029x over previous
import jax
import jax.numpy as jnp
from jax import lax
from jax.experimental import pallas as pl
from jax.experimental.pallas import tpu as pltpu
from jax.experimental.pallas import tpu_sc as plsc

BATCH = 4
SEQ = 2048
EMBED = 1024
N = BATCH * SEQ

NUM_CORES = 2
NUM_SUBCORES = 16
NW = NUM_CORES * NUM_SUBCORES
POS_PER_W = SEQ // NW
ROWS_PER_W = POS_PER_W * BATCH
CHUNK = 16
CHUNKS_PER_B = POS_PER_W // CHUNK
NCHUNK = CHUNKS_PER_B * BATCH
LANES = 16
VECS_PER_ROW = EMBED // LANES


NBUF = 3


def _sc_body(x_hbm, tok_hbm, pos_hbm, out_hbm,
             idx_v, pos_v, tok0, tok1, tok2, g0, g1, g2, o0, o1, o2, psem):
    wid = lax.axis_index("s") * NUM_CORES + lax.axis_index("c")
    pos0 = wid * POS_PER_W

    for b in range(BATCH):
        pltpu.sync_copy(x_hbm.at[b, pl.ds(pos0, POS_PER_W)],
                        idx_v.at[pl.ds(b * POS_PER_W, POS_PER_W)])

    pos_cp = pltpu.async_copy(pos_hbm.at[pl.ds(pos0, POS_PER_W)], pos_v, psem)

    bufs = (tok0, tok1, tok2)
    gsems = (g0, g1, g2)
    osems = (o0, o1, o2)

    def idx_slice(c):
        return idx_v.at[pl.ds(c * CHUNK, CHUNK)]

    def out_slice(c):
        b, sub = c // CHUNKS_PER_B, c % CHUNKS_PER_B
        return out_hbm.at[b, pl.ds(pos0 + sub * CHUNK, CHUNK)]

    gathers = [None] * NCHUNK
    scatters = [None] * NCHUNK
    for c in range(NBUF - 1):
        gathers[c] = pltpu.async_copy(
            tok_hbm.at[idx_slice(c)], bufs[c % NBUF], gsems[c % NBUF])
    pos_cp.wait()

    for c in range(NCHUNK):
        k = c % NBUF
        buf = bufs[k]
        gathers[c].wait()
        nxt = c + NBUF - 1
        if nxt < NCHUNK:
            if nxt >= NBUF:
                scatters[nxt - NBUF].wait()
            gathers[nxt] = pltpu.async_copy(
                tok_hbm.at[idx_slice(nxt)], bufs[nxt % NBUF], gsems[nxt % NBUF])

        prow = (c % CHUNKS_PER_B) * CHUNK

        def add_row(r, carry):
            for j in range(VECS_PER_ROW):
                sl = pl.ds(j * LANES, LANES)
                plsc.addupdate(buf.at[r, sl], pos_v[prow + r, sl])
            return carry

        lax.fori_loop(0, CHUNK, add_row, 0)

        scatters[c] = pltpu.async_copy(buf, out_slice(c), osems[k])

    for c in range(NCHUNK - min(NBUF, NCHUNK), NCHUNK):
        scatters[c].wait()


@jax.jit
def kernel(x, token_table, pos_table):
    mesh = plsc.VectorSubcoreMesh(
        core_axis_name="c", subcore_axis_name="s",
        num_cores=NUM_CORES, num_subcores=NUM_SUBCORES,
    )
    return pl.kernel(
        _sc_body,
        out_type=jax.ShapeDtypeStruct((BATCH, SEQ, EMBED), jnp.float32),
        mesh=mesh,
        scratch_types=[
            pltpu.VMEM((ROWS_PER_W,), jnp.int32),
            pltpu.VMEM((POS_PER_W, EMBED), jnp.float32),
            pltpu.VMEM((CHUNK, EMBED), jnp.float32),
            pltpu.VMEM((CHUNK, EMBED), jnp.float32),
            pltpu.VMEM((CHUNK, EMBED), jnp.float32),
            pltpu.SemaphoreType.DMA,
            pltpu.SemaphoreType.DMA,
            pltpu.SemaphoreType.DMA,
            pltpu.SemaphoreType.DMA,
            pltpu.SemaphoreType.DMA,
            pltpu.SemaphoreType.DMA,
            pltpu.SemaphoreType.DMA,
        ],
    )(x, token_table, pos_table)


# speedup vs baseline: 1.2007x; 1.1972x over previous
import jax
import jax.numpy as jnp
from jax import lax
from jax.experimental import pallas as pl
from jax.experimental.pallas import tpu as pltpu
from jax.experimental.pallas import tpu_sc as plsc

BATCH = 4
SEQ = 2048
EMBED = 1024
N = BATCH * SEQ

NUM_CORES = 2
NUM_SUBCORES = 16
NW = NUM_CORES * NUM_SUBCORES
POS_PER_W = SEQ // NW
ROWS_PER_W = POS_PER_W * BATCH
CHUNK = 16
NSUB = POS_PER_W // CHUNK
NCHUNK = NSUB * BATCH
LANES = 16
VECS_PER_ROW = EMBED // LANES

NBUF = 5
NPOS = 2


def _sc_body(x_hbm, tok_hbm, pos_hbm, out_hbm,
             idx_v, pos0_v, pos1_v,
             tok0, tok1, tok2, tok3, tok4,
             g0, g1, g2, g3, g4,
             o0, o1, o2, o3, o4,
             p0, p1):
    wid = lax.axis_index("s") * NUM_CORES + lax.axis_index("c")
    pos0 = wid * POS_PER_W

    for b in range(BATCH):
        pltpu.sync_copy(x_hbm.at[b, pl.ds(pos0, POS_PER_W)],
                        idx_v.at[pl.ds(b * POS_PER_W, POS_PER_W)])

    bufs = (tok0, tok1, tok2, tok3, tok4)
    gsems = (g0, g1, g2, g3, g4)
    osems = (o0, o1, o2, o3, o4)
    pos_bufs = (pos0_v, pos1_v)
    psems = (p0, p1)

    def idx_slice(c):
        sub, b = c // BATCH, c % BATCH
        return idx_v.at[pl.ds(b * POS_PER_W + sub * CHUNK, CHUNK)]

    def out_slice(c):
        sub, b = c // BATCH, c % BATCH
        return out_hbm.at[b, pl.ds(pos0 + sub * CHUNK, CHUNK)]

    pos_cps = [None] * NSUB
    for s in range(NPOS):
        pos_cps[s] = pltpu.async_copy(
            pos_hbm.at[pl.ds(pos0 + s * CHUNK, CHUNK)],
            pos_bufs[s % NPOS], psems[s % NPOS])

    gathers = [None] * NCHUNK
    scatters = [None] * NCHUNK
    for c in range(NBUF - 1):
        gathers[c] = pltpu.async_copy(
            tok_hbm.at[idx_slice(c)], bufs[c % NBUF], gsems[c % NBUF])

    for c in range(NCHUNK):
        k = c % NBUF
        buf = bufs[k]
        sub = c // BATCH
        if c % BATCH == 0:
            pos_cps[sub].wait()
        pv = pos_bufs[sub % NPOS]
        gathers[c].wait()

        nxt = c + NBUF - 1
        if nxt < NCHUNK:
            if nxt >= NBUF:
                scatters[nxt - NBUF].wait()
            gathers[nxt] = pltpu.async_copy(
                tok_hbm.at[idx_slice(nxt)], bufs[nxt % NBUF], gsems[nxt % NBUF])

        def add_row(r, carry):
            for j in range(VECS_PER_ROW):
                sl = pl.ds(j * LANES, LANES)
                plsc.addupdate(buf.at[r, sl], pv[r, sl])
            return carry

        lax.fori_loop(0, CHUNK, add_row, 0)

        scatters[c] = pltpu.async_copy(buf, out_slice(c), osems[k])

        if c % BATCH == BATCH - 1:
            ns = sub + NPOS
            if ns < NSUB:
                pos_cps[ns] = pltpu.async_copy(
                    pos_hbm.at[pl.ds(pos0 + ns * CHUNK, CHUNK)],
                    pos_bufs[ns % NPOS], psems[ns % NPOS])

    for c in range(NCHUNK - min(NBUF, NCHUNK), NCHUNK):
        scatters[c].wait()


@jax.jit
def kernel(x, token_table, pos_table):
    mesh = plsc.VectorSubcoreMesh(
        core_axis_name="c", subcore_axis_name="s",
        num_cores=NUM_CORES, num_subcores=NUM_SUBCORES,
    )
    return pl.kernel(
        _sc_body,
        out_type=jax.ShapeDtypeStruct((BATCH, SEQ, EMBED), jnp.float32),
        mesh=mesh,
        scratch_types=[
            pltpu.VMEM((ROWS_PER_W,), jnp.int32),
            pltpu.VMEM((CHUNK, EMBED), jnp.float32),
            pltpu.VMEM((CHUNK, EMBED), jnp.float32),
        ] + [pltpu.VMEM((CHUNK, EMBED), jnp.float32)] * NBUF
        + [pltpu.SemaphoreType.DMA] * (2 * NBUF + NPOS),
    )(x, token_table, pos_table)


# speedup vs baseline: 1.2201x; 1.0162x over previous
import jax
import jax.numpy as jnp
from jax import lax
from jax.experimental import pallas as pl
from jax.experimental.pallas import tpu as pltpu
from jax.experimental.pallas import tpu_sc as plsc

BATCH = 4
SEQ = 2048
EMBED = 1024
N = BATCH * SEQ

NUM_CORES = 2
NUM_SUBCORES = 16
NW = NUM_CORES * NUM_SUBCORES
POS_PER_W = SEQ // NW
ROWS_PER_W = POS_PER_W * BATCH
CHUNK = 16
NSUB = POS_PER_W // CHUNK
NCHUNK = NSUB * BATCH
LANES = 16
VECS_PER_ROW = EMBED // LANES

NBUF = 5
NPOS = 2


def _sc_body(x_hbm, tok_hbm, pos_hbm, out_hbm, idx_v, *scratch):
    pos_bufs = scratch[:NPOS]
    bufs = scratch[NPOS:NPOS + NBUF]
    gsems = scratch[NPOS + NBUF:NPOS + 2 * NBUF]
    osems = scratch[NPOS + 2 * NBUF:NPOS + 3 * NBUF]
    psems = scratch[NPOS + 3 * NBUF:NPOS + 3 * NBUF + NPOS]
    isem = scratch[NPOS + 3 * NBUF + NPOS]
    wid = lax.axis_index("s") * NUM_CORES + lax.axis_index("c")
    pos0 = wid * POS_PER_W

    idx_cps = [
        pltpu.async_copy(x_hbm.at[b, pl.ds(pos0, POS_PER_W)],
                         idx_v.at[b], isem)
        for b in range(BATCH)
    ]

    def idx_slice(c):
        sub, b = c // BATCH, c % BATCH
        return idx_v.at[b, pl.ds(sub * CHUNK, CHUNK)]

    def out_slice(c):
        sub, b = c // BATCH, c % BATCH
        return out_hbm.at[b, pl.ds(pos0 + sub * CHUNK, CHUNK)]

    pos_cps = [None] * NSUB
    for s in range(NPOS):
        pos_cps[s] = pltpu.async_copy(
            pos_hbm.at[pl.ds(pos0 + s * CHUNK, CHUNK)],
            pos_bufs[s % NPOS], psems[s % NPOS])

    gathers = [None] * NCHUNK
    scatters = [None] * NCHUNK
    for cp in idx_cps:
        cp.wait()
    for c in range(NBUF - 1):
        gathers[c] = pltpu.async_copy(
            tok_hbm.at[idx_slice(c)], bufs[c % NBUF], gsems[c % NBUF])

    for c in range(NCHUNK):
        k = c % NBUF
        buf = bufs[k]
        sub = c // BATCH
        if c % BATCH == 0:
            pos_cps[sub].wait()
        pv = pos_bufs[sub % NPOS]
        gathers[c].wait()

        nxt = c + NBUF - 1
        if nxt < NCHUNK:
            if nxt >= NBUF:
                scatters[nxt - NBUF].wait()
            gathers[nxt] = pltpu.async_copy(
                tok_hbm.at[idx_slice(nxt)], bufs[nxt % NBUF], gsems[nxt % NBUF])

        def add_row(r, carry):
            for j in range(VECS_PER_ROW):
                sl = pl.ds(j * LANES, LANES)
                plsc.addupdate(buf.at[r, sl], pv[r, sl])
            return carry

        lax.fori_loop(0, CHUNK, add_row, 0)

        scatters[c] = pltpu.async_copy(buf, out_slice(c), osems[k])

        if c % BATCH == BATCH - 1:
            ns = sub + NPOS
            if ns < NSUB:
                pos_cps[ns] = pltpu.async_copy(
                    pos_hbm.at[pl.ds(pos0 + ns * CHUNK, CHUNK)],
                    pos_bufs[ns % NPOS], psems[ns % NPOS])

    for c in range(NCHUNK - min(NBUF, NCHUNK), NCHUNK):
        scatters[c].wait()


@jax.jit
def kernel(x, token_table, pos_table):
    mesh = plsc.VectorSubcoreMesh(
        core_axis_name="c", subcore_axis_name="s",
        num_cores=NUM_CORES, num_subcores=NUM_SUBCORES,
    )
    return pl.kernel(
        _sc_body,
        out_type=jax.ShapeDtypeStruct((BATCH, SEQ, EMBED), jnp.float32),
        mesh=mesh,
        scratch_types=[pltpu.VMEM((BATCH, POS_PER_W), jnp.int32)]
        + [pltpu.VMEM((CHUNK, EMBED), jnp.float32)] * (NPOS + NBUF)
        + [pltpu.SemaphoreType.DMA] * (2 * NBUF + NPOS + 1),
    )(x, token_table, pos_table)
